# baseline (speedup 1.0000x reference)
import functools

import jax
import jax.numpy as jnp
from jax import lax
from jax.experimental import pallas as pl
from jax.experimental.pallas import tpu as pltpu
from jax.experimental.pallas import tpu_sc as plsc

HIDDEN_DIM = 128
VOCAB_ROWS = 118
N = 100000

_NC = 2
_NS = 16
_NW = _NC * _NS

_PER_W = 3136
_CHUNK = 392
_NCHUNK = _PER_W // _CHUNK
_SROWS = 248
_LROWS = _CHUNK - _SROWS
_PER_W_LAST = N - (_NW - 1) * _PER_W
_TAIL = _PER_W_LAST - (_NCHUNK - 1) * _CHUNK
_NJ = HIDDEN_DIM // 16


@functools.partial(
    pl.kernel,
    mesh=plsc.VectorSubcoreMesh(core_axis_name="c", subcore_axis_name="s"),
    out_type=jax.ShapeDtypeStruct((N, HIDDEN_DIM), jnp.float32),
    scratch_types=[
        pltpu.VMEM((_PER_W,), jnp.int32),
        pltpu.VMEM((VOCAB_ROWS, HIDDEN_DIM), jnp.float32),
        pltpu.VMEM((_CHUNK, HIDDEN_DIM), jnp.float32),
        pltpu.VMEM((_CHUNK, HIDDEN_DIM), jnp.float32),
        pltpu.VMEM_SHARED((VOCAB_ROWS, HIDDEN_DIM), jnp.float32),
        pltpu.SemaphoreType.DMA,
        pltpu.SemaphoreType.DMA,
        pltpu.SemaphoreType.DMA,
        pltpu.SemaphoreType.DMA,
        pltpu.SemaphoreType.DMA,
        pltpu.SemaphoreType.DMA,
    ],
)
def _embedding_gather(table_hbm, idx_hbm, out_hbm, idx_all, table_loc,
                      rows0, rows1, table_sh, tsem, lsem,
                      gsem0, gsem1, osem0, osem1):
    wid = lax.axis_index("s") * _NC + lax.axis_index("c")
    base = wid * _PER_W
    rows = (rows0, rows1)
    gsem = (gsem0, gsem1)
    osem = (osem0, osem1)
    sid = lax.axis_index("s")

    tloc = pltpu.make_async_copy(table_hbm, table_loc, lsem)
    tloc.start()
    tsh = pltpu.make_async_copy(table_hbm, table_sh, tsem)

    @pl.when(sid == 0)
    def _():
        tsh.start()

    @pl.when(wid < _NW - 1)
    def _():
        pltpu.sync_copy(idx_hbm.at[pl.ds(base, _PER_W)], idx_all)

    @pl.when(wid == _NW - 1)
    def _():
        pltpu.sync_copy(idx_hbm.at[pl.ds(base, _PER_W_LAST)],
                        idx_all.at[pl.ds(0, _PER_W_LAST)])

    tloc.wait()

    @pl.when(sid == 0)
    def _():
        tsh.wait()

    plsc.subcore_barrier()

    def local_rows(k, b):
        roff = k * _CHUNK + _SROWS

        def grp(i, carry):
            r0 = i * 16
            iv = idx_all[pl.ds(roff + r0, 16)]
            ss = [iv[u] for u in range(16)]

            def loads(p):
                return [[table_loc[ss[2 * p + t], pl.ds(16 * j, 16)]
                         for j in range(_NJ)] for t in range(2)]

            def stores(p, vals):
                for t in range(2):
                    for j in range(_NJ):
                        rows[b][_SROWS + r0 + 2 * p + t,
                                pl.ds(16 * j, 16)] = vals[t][j]

            cur = loads(0)
            for p in range(1, 8):
                nxt = loads(p)
                stores(p - 1, cur)
                cur = nxt
            stores(7, cur)
            return carry

        lax.fori_loop(0, _LROWS // 16, grp, 0)

    def chunk(k, b):
        g = pltpu.async_copy(
            table_sh.at[idx_all.at[pl.ds(k * _CHUNK, _SROWS)]],
            rows[b].at[pl.ds(0, _SROWS)], gsem[b])
        local_rows(k, b)
        g.wait()
        return pltpu.async_copy(
            rows[b], out_hbm.at[pl.ds(base + k * _CHUNK, _CHUNK)], osem[b])

    stores = [None, None]
    for k in range(_NCHUNK - 1):
        b = k & 1
        if stores[b] is not None:
            stores[b].wait()
        stores[b] = chunk(k, b)

    stores[1].wait()

    @pl.when(wid < _NW - 1)
    def _():
        chunk(_NCHUNK - 1, 1).wait()

    @pl.when(wid == _NW - 1)
    def _():
        last = (_NCHUNK - 1) * _CHUNK
        pltpu.async_copy(
            table_sh.at[idx_all.at[pl.ds(last, _TAIL)]],
            rows1.at[pl.ds(0, _TAIL)], gsem1).wait()
        pltpu.async_copy(rows1.at[pl.ds(0, _TAIL)],
                         out_hbm.at[pl.ds(base + last, _TAIL)], osem1).wait()

    stores[0].wait()


def kernel(atom_num, table):
    idx = atom_num.astype(jnp.int32)
    return _embedding_gather(table, idx)


# speedup vs baseline: 1.0539x; 1.0539x over previous
import functools

import jax
import jax.numpy as jnp
from jax import lax
from jax.experimental import pallas as pl
from jax.experimental.pallas import tpu as pltpu
from jax.experimental.pallas import tpu_sc as plsc

HIDDEN_DIM = 128
VOCAB_ROWS = 118
N = 100000

_NC = 2
_NS = 16
_NW = _NC * _NS

_PER_W = 3136
_CHUNK = 224
_NCHUNK = _PER_W // _CHUNK
_PER_W_LAST = N - (_NW - 1) * _PER_W
_NFULL_LAST = _PER_W_LAST // _CHUNK
_TAIL = _PER_W_LAST - _NFULL_LAST * _CHUNK


@functools.partial(
    pl.kernel,
    mesh=plsc.VectorSubcoreMesh(core_axis_name="c", subcore_axis_name="s"),
    out_type=jax.ShapeDtypeStruct((N, HIDDEN_DIM), jnp.float32),
    scratch_types=[
        pltpu.VMEM((_PER_W,), jnp.int32),
        pltpu.VMEM((_CHUNK, HIDDEN_DIM), jnp.float32),
        pltpu.VMEM((_CHUNK, HIDDEN_DIM), jnp.float32),
        pltpu.VMEM_SHARED((VOCAB_ROWS, HIDDEN_DIM), jnp.float32),
        pltpu.SemaphoreType.DMA,
        pltpu.SemaphoreType.DMA,
        pltpu.SemaphoreType.DMA,
        pltpu.SemaphoreType.DMA,
        pltpu.SemaphoreType.DMA,
    ],
)
def _embedding_gather(table_hbm, idx_hbm, out_hbm, idx_all, rows0, rows1,
                      table_sh, tsem, gsem0, gsem1, osem0, osem1):
    wid = lax.axis_index("s") * _NC + lax.axis_index("c")
    base = wid * _PER_W
    rows = (rows0, rows1)
    gsem = (gsem0, gsem1)
    osem = (osem0, osem1)

    sid = lax.axis_index("s")
    tl = pltpu.make_async_copy(table_hbm, table_sh, tsem)

    @pl.when(sid == 0)
    def _():
        tl.start()

    @pl.when(wid < _NW - 1)
    def _():
        pltpu.sync_copy(idx_hbm.at[pl.ds(base, _PER_W)], idx_all)

    @pl.when(wid == _NW - 1)
    def _():
        pltpu.sync_copy(idx_hbm.at[pl.ds(base, _PER_W_LAST)],
                        idx_all.at[pl.ds(0, _PER_W_LAST)])

    @pl.when(sid == 0)
    def _():
        tl.wait()

    plsc.subcore_barrier()

    def chunk(k, nrows, b):
        pltpu.async_copy(
            table_sh.at[idx_all.at[pl.ds(k * _CHUNK, nrows)]],
            rows[b].at[pl.ds(0, nrows)], gsem[b]).wait()
        return pltpu.async_copy(
            rows[b].at[pl.ds(0, nrows)],
            out_hbm.at[pl.ds(base + k * _CHUNK, nrows)], osem[b])

    stores = [None, None]
    for k in range(_NFULL_LAST):
        b = k & 1
        if stores[b] is not None:
            stores[b].wait()
        stores[b] = chunk(k, _CHUNK, b)

    stores[0].wait()
    stores[1].wait()

    @pl.when(wid < _NW - 1)
    def _():
        st_a = chunk(_NFULL_LAST, _CHUNK, 0)
        st_b = chunk(_NFULL_LAST + 1, _CHUNK, 1)
        st_a.wait()
        st_b.wait()

    @pl.when(wid == _NW - 1)
    def _():
        chunk(_NFULL_LAST, _TAIL, 0).wait()


def kernel(atom_num, table):
    idx = atom_num.astype(jnp.int32)
    return _embedding_gather(table, idx)


# speedup vs baseline: 1.0788x; 1.0236x over previous
import functools

import jax
import jax.numpy as jnp
from jax import lax
from jax.experimental import pallas as pl
from jax.experimental.pallas import tpu as pltpu
from jax.experimental.pallas import tpu_sc as plsc

HIDDEN_DIM = 128
VOCAB_ROWS = 118
N = 100000

_NC = 2
_NS = 16
_NW = _NC * _NS

_PER_W = 3136
_CHUNK = 392
_NCHUNK = _PER_W // _CHUNK
_PER_W_LAST = N - (_NW - 1) * _PER_W
_TAIL = _PER_W_LAST - (_NCHUNK - 1) * _CHUNK


@functools.partial(
    pl.kernel,
    mesh=plsc.VectorSubcoreMesh(core_axis_name="c", subcore_axis_name="s"),
    out_type=jax.ShapeDtypeStruct((N, HIDDEN_DIM), jnp.float32),
    scratch_types=[
        pltpu.VMEM((_PER_W,), jnp.int32),
        pltpu.VMEM((_CHUNK, HIDDEN_DIM), jnp.float32),
        pltpu.VMEM((_CHUNK, HIDDEN_DIM), jnp.float32),
        pltpu.VMEM_SHARED((VOCAB_ROWS, HIDDEN_DIM), jnp.float32),
        pltpu.SemaphoreType.DMA,
        pltpu.SemaphoreType.DMA,
        pltpu.SemaphoreType.DMA,
        pltpu.SemaphoreType.DMA,
        pltpu.SemaphoreType.DMA,
    ],
)
def _embedding_gather(table_hbm, idx_hbm, out_hbm, idx_all, rows0, rows1,
                      table_sh, tsem, gsem0, gsem1, osem0, osem1):
    wid = lax.axis_index("s") * _NC + lax.axis_index("c")
    base = wid * _PER_W
    rows = (rows0, rows1)
    gsem = (gsem0, gsem1)
    osem = (osem0, osem1)

    sid = lax.axis_index("s")
    tl = pltpu.make_async_copy(table_hbm, table_sh, tsem)

    @pl.when(sid == 0)
    def _():
        tl.start()

    @pl.when(wid < _NW - 1)
    def _():
        pltpu.sync_copy(idx_hbm.at[pl.ds(base, _PER_W)], idx_all)

    @pl.when(wid == _NW - 1)
    def _():
        pltpu.sync_copy(idx_hbm.at[pl.ds(base, _PER_W_LAST)],
                        idx_all.at[pl.ds(0, _PER_W_LAST)])

    @pl.when(sid == 0)
    def _():
        tl.wait()

    plsc.subcore_barrier()

    def chunk(k, nrows, b):
        pltpu.async_copy(
            table_sh.at[idx_all.at[pl.ds(k * _CHUNK, nrows)]],
            rows[b].at[pl.ds(0, nrows)], gsem[b]).wait()
        return pltpu.async_copy(
            rows[b].at[pl.ds(0, nrows)],
            out_hbm.at[pl.ds(base + k * _CHUNK, nrows)], osem[b])

    stores = [None, None]
    for k in range(_NCHUNK - 1):
        b = k & 1
        if stores[b] is not None:
            stores[b].wait()
        stores[b] = chunk(k, _CHUNK, b)

    stores[1].wait()

    @pl.when(wid < _NW - 1)
    def _():
        chunk(_NCHUNK - 1, _CHUNK, 1).wait()

    @pl.when(wid == _NW - 1)
    def _():
        chunk(_NCHUNK - 1, _TAIL, 1).wait()

    stores[0].wait()


def kernel(atom_num, table):
    idx = atom_num.astype(jnp.int32)
    return _embedding_gather(table, idx)
